# Initial kernel scaffold; baseline (speedup 1.0000x reference)
import functools

import jax
import jax.numpy as jnp
from jax import lax
from jax.experimental import pallas as pl
from jax.experimental.pallas import tpu as pltpu
from jax.experimental.pallas import tpu_sc as plsc

N = 10000
E = 320000
D = 128
N_PAD = 10240

NC = 2
NS = 16
NW = NC * NS
EP = E // NW
CH = 80
NCHUNK = EP // CH
ZR = N_PAD // NS


def _sc_sparse_accumulate(x, src, dst, vals):
    mesh = plsc.VectorSubcoreMesh(core_axis_name="c", subcore_axis_name="s")

    @functools.partial(
        pl.kernel,
        out_type=jax.ShapeDtypeStruct((NC * N_PAD, D), jnp.float32),
        mesh=mesh,
        scratch_types=[
            pltpu.VMEM((CH,), jnp.int32),
            pltpu.VMEM((CH,), jnp.int32),
            pltpu.VMEM((CH,), jnp.float32),
            pltpu.VMEM((CH, D), jnp.float32),
            pltpu.VMEM_SHARED((N_PAD, D), jnp.float32),
            pltpu.SemaphoreType.DMA,
        ],
    )
    def sc_kernel(x_hbm, src_hbm, dst_hbm, val_hbm, out_hbm,
                  src_v, dst_v, val_v, rows_v, acc_sh, sem):
        c = lax.axis_index("c")
        s = lax.axis_index("s")
        wid = s * NC + c

        def zero_body(i, carry):
            z = jnp.zeros((16,), jnp.float32)
            for j in range(D // 16):
                rows_v[i, pl.ds(j * 16, 16)] = z
            return carry
        lax.fori_loop(0, CH, zero_body, 0)
        for k in range(ZR // CH):
            pltpu.sync_copy(rows_v, acc_sh.at[pl.ds(s * ZR + k * CH, CH)])
        plsc.subcore_barrier()

        base = wid * EP

        def chunk_body(ci, carry):
            off = base + ci * CH
            pltpu.sync_copy(src_hbm.at[pl.ds(off, CH)], src_v)
            pltpu.sync_copy(dst_hbm.at[pl.ds(off, CH)], dst_v)
            pltpu.sync_copy(val_hbm.at[pl.ds(off, CH)], val_v)
            pltpu.async_copy(x_hbm.at[src_v], rows_v, sem).wait()

            def scale_body(e, inner):
                v = val_v[e]
                vv = jnp.full((16,), v, jnp.float32)
                for j in range(D // 16):
                    sl = pl.ds(j * 16, 16)
                    rows_v[e, sl] = rows_v[e, sl] * vv
                return inner
            lax.fori_loop(0, CH, scale_body, 0)

            pltpu.sync_copy(rows_v, acc_sh.at[dst_v], add=True)
            return carry
        lax.fori_loop(0, NCHUNK, chunk_body, 0)

        plsc.subcore_barrier()
        pltpu.sync_copy(acc_sh.at[pl.ds(s * ZR, ZR)],
                        out_hbm.at[pl.ds(c * N_PAD + s * ZR, ZR)])

    return sc_kernel(x, src, dst, vals)


def _tc_epilogue(acc0, acc1, W, b2d):
    BLK = 512

    def body(a0_ref, a1_ref, w_ref, b_ref, o_ref):
        a = a0_ref[...] + a1_ref[...]
        y = jnp.dot(a, w_ref[...], preferred_element_type=jnp.float32)
        o_ref[...] = jnp.maximum(y + b_ref[...], 0.0)

    return pl.pallas_call(
        body,
        grid=(N_PAD // BLK,),
        in_specs=[
            pl.BlockSpec((BLK, D), lambda i: (i, 0)),
            pl.BlockSpec((BLK, D), lambda i: (i, 0)),
            pl.BlockSpec((D, D), lambda i: (0, 0)),
            pl.BlockSpec((1, D), lambda i: (0, 0)),
        ],
        out_specs=pl.BlockSpec((BLK, D), lambda i: (i, 0)),
        out_shape=jax.ShapeDtypeStruct((N_PAD, D), jnp.float32),
    )(acc0, acc1, W, b2d)


def kernel(x, adj_indices, adj_values, W, b):
    dst = adj_indices[0]
    src = adj_indices[1]
    acc = _sc_sparse_accumulate(x, src, dst, adj_values)
    out = _tc_epilogue(acc[:N_PAD], acc[N_PAD:], W, b.reshape(1, D))
    return out[:N]


# speedup vs baseline: 4.3941x; 4.3941x over previous
import functools

import jax
import jax.numpy as jnp
from jax import lax
from jax.experimental import pallas as pl
from jax.experimental.pallas import tpu as pltpu
from jax.experimental.pallas import tpu_sc as plsc

N = 10000
E = 320000
D = 128
N_PAD = 10240

NC = 2
NS = 16
NW = NC * NS
EP = E // NW
CH = 80
NCHUNK = EP // CH
ZR = N_PAD // NS


def _sc_sparse_accumulate(x, src, dst, vals):
    mesh = plsc.VectorSubcoreMesh(core_axis_name="c", subcore_axis_name="s")

    @functools.partial(
        pl.kernel,
        out_type=jax.ShapeDtypeStruct((NC * N_PAD, D), jnp.float32),
        mesh=mesh,
        scratch_types=[
            pltpu.VMEM((CH,), jnp.int32),
            pltpu.VMEM((CH,), jnp.int32),
            pltpu.VMEM((CH,), jnp.float32),
            pltpu.VMEM((CH, D), jnp.float32),
            pltpu.VMEM_SHARED((N_PAD, D), jnp.float32),
            pltpu.SemaphoreType.DMA,
        ],
    )
    def sc_kernel(x_hbm, src_hbm, dst_hbm, val_hbm, out_hbm,
                  src_v, dst_v, val_v, rows_v, acc_sh, sem):
        c = lax.axis_index("c")
        s = lax.axis_index("s")
        wid = s * NC + c

        def zero_body(i, carry):
            z = jnp.zeros((16,), jnp.float32)
            for j in range(D // 16):
                rows_v[i, pl.ds(j * 16, 16)] = z
            return carry
        lax.fori_loop(0, CH, zero_body, 0)
        for k in range(ZR // CH):
            pltpu.sync_copy(rows_v, acc_sh.at[pl.ds(s * ZR + k * CH, CH)])
        plsc.subcore_barrier()

        base = wid * EP

        def chunk_body(ci, carry):
            off = base + ci * CH
            pltpu.sync_copy(src_hbm.at[pl.ds(off, CH)], src_v)
            pltpu.sync_copy(dst_hbm.at[pl.ds(off, CH)], dst_v)
            pltpu.sync_copy(val_hbm.at[pl.ds(off, CH)], val_v)
            pltpu.async_copy(x_hbm.at[src_v], rows_v, sem).wait()

            def scale_group(g, inner):
                vvec = val_v[pl.ds(g * 16, 16)]
                for l in range(16):
                    lane = jnp.full((16,), l, jnp.int32)
                    vv = vvec.at[lane].get(mode="promise_in_bounds")
                    e = g * 16 + l
                    for j in range(D // 16):
                        sl = pl.ds(j * 16, 16)
                        rows_v[e, sl] = rows_v[e, sl] * vv
                return inner
            lax.fori_loop(0, CH // 16, scale_group, 0)

            pltpu.sync_copy(rows_v, acc_sh.at[dst_v], add=True)
            return carry
        lax.fori_loop(0, NCHUNK, chunk_body, 0)

        plsc.subcore_barrier()
        pltpu.sync_copy(acc_sh.at[pl.ds(s * ZR, ZR)],
                        out_hbm.at[pl.ds(c * N_PAD + s * ZR, ZR)])

    return sc_kernel(x, src, dst, vals)


def _tc_epilogue(acc0, acc1, W, b2d):
    BLK = 512

    def body(a0_ref, a1_ref, w_ref, b_ref, o_ref):
        a = a0_ref[...] + a1_ref[...]
        y = jnp.dot(a, w_ref[...], preferred_element_type=jnp.float32)
        o_ref[...] = jnp.maximum(y + b_ref[...], 0.0)

    return pl.pallas_call(
        body,
        grid=(N_PAD // BLK,),
        in_specs=[
            pl.BlockSpec((BLK, D), lambda i: (i, 0)),
            pl.BlockSpec((BLK, D), lambda i: (i, 0)),
            pl.BlockSpec((D, D), lambda i: (0, 0)),
            pl.BlockSpec((1, D), lambda i: (0, 0)),
        ],
        out_specs=pl.BlockSpec((BLK, D), lambda i: (i, 0)),
        out_shape=jax.ShapeDtypeStruct((N_PAD, D), jnp.float32),
    )(acc0, acc1, W, b2d)


def kernel(x, adj_indices, adj_values, W, b):
    dst = adj_indices[0]
    src = adj_indices[1]
    acc = _sc_sparse_accumulate(x, src, dst, adj_values)
    out = _tc_epilogue(acc[:N_PAD], acc[N_PAD:], W, b.reshape(1, D))
    return out[:N]


# speedup vs baseline: 6.2367x; 1.4193x over previous
import functools

import jax
import jax.numpy as jnp
from jax import lax
from jax.experimental import pallas as pl
from jax.experimental.pallas import tpu as pltpu
from jax.experimental.pallas import tpu_sc as plsc

N = 10000
E = 320000
D = 128
N_PAD = 10240

NC = 2
NS = 16
NW = NC * NS
CH = 80
NRB = 3
NMB = 6
NCHUNK = 126
EP = NCHUNK * CH
E_PAD = NW * EP
ZR = N_PAD // NS
NITER = NCHUNK // NMB


def _sc_sparse_accumulate(x, src, dst, vals):
    mesh = plsc.VectorSubcoreMesh(core_axis_name="c", subcore_axis_name="s")

    scratch = (
        [pltpu.VMEM((CH,), jnp.int32) for _ in range(NMB)]
        + [pltpu.VMEM((CH,), jnp.int32) for _ in range(NMB)]
        + [pltpu.VMEM((CH,), jnp.float32) for _ in range(NMB)]
        + [pltpu.VMEM((CH, D), jnp.float32) for _ in range(NRB)]
        + [pltpu.VMEM_SHARED((N_PAD, D), jnp.float32)]
        + [pltpu.SemaphoreType.DMA for _ in range(NMB)]
        + [pltpu.SemaphoreType.DMA for _ in range(NRB)]
        + [pltpu.SemaphoreType.DMA for _ in range(NRB)]
    )

    @functools.partial(
        pl.kernel,
        out_type=jax.ShapeDtypeStruct((NC * N_PAD, D), jnp.float32),
        mesh=mesh,
        scratch_types=scratch,
    )
    def sc_kernel(x_hbm, src_hbm, dst_hbm, val_hbm, out_hbm, *refs):
        msrc = refs[0:NMB]
        mdst = refs[NMB:2 * NMB]
        mval = refs[2 * NMB:3 * NMB]
        rows = refs[3 * NMB:3 * NMB + NRB]
        acc_sh = refs[3 * NMB + NRB]
        msem = refs[3 * NMB + NRB + 1:3 * NMB + NRB + 1 + NMB]
        gsem = refs[3 * NMB + NRB + 1 + NMB:3 * NMB + NRB + 1 + NMB + NRB]
        ssem = refs[3 * NMB + NRB + 1 + NMB + NRB:]

        c_ax = lax.axis_index("c")
        s_ax = lax.axis_index("s")
        wid = s_ax * NC + c_ax
        base = wid * EP

        def start_meta(ch, m):
            off = base + ch * CH
            pltpu.async_copy(src_hbm.at[pl.ds(off, CH)], msrc[m], msem[m])
            pltpu.async_copy(dst_hbm.at[pl.ds(off, CH)], mdst[m], msem[m])
            pltpu.async_copy(val_hbm.at[pl.ds(off, CH)], mval[m], msem[m])

        def wait_meta(ch, m):
            off = base + ch * CH
            pltpu.make_async_copy(src_hbm.at[pl.ds(off, CH)], msrc[m],
                                  msem[m]).wait()
            pltpu.make_async_copy(dst_hbm.at[pl.ds(off, CH)], mdst[m],
                                  msem[m]).wait()
            pltpu.make_async_copy(val_hbm.at[pl.ds(off, CH)], mval[m],
                                  msem[m]).wait()

        def start_gather(m, r):
            pltpu.async_copy(x_hbm.at[msrc[m]], rows[r], gsem[r])

        def wait_gather(m, r):
            pltpu.make_async_copy(x_hbm.at[msrc[m]], rows[r], gsem[r]).wait()

        def start_scatter(m, r):
            pltpu.async_copy(rows[r], acc_sh.at[mdst[m]], ssem[r], add=True)

        def wait_scatter(m, r):
            pltpu.make_async_copy(rows[r], acc_sh.at[mdst[m]], ssem[r]).wait()

        def scale(m, r):
            @pl.loop(0, CH // 16)
            def scale_group(g):
                for l in range(16):
                    lane = jnp.full((16,), l, jnp.int32)
                    vv = mval[m][pl.ds(g * 16, 16)].at[lane].get(
                        mode="promise_in_bounds")
                    e = g * 16 + l
                    for j in range(D // 16):
                        sl = pl.ds(j * 16, 16)
                        rows[r][e, sl] = rows[r][e, sl] * vv

        for ch in range(4):
            start_meta(ch, ch)
        for ch in range(2):
            wait_meta(ch, ch)
            start_gather(ch, ch)

        @pl.loop(0, CH)
        def zero_body(i):
            z = jnp.zeros((16,), jnp.float32)
            for j in range(D // 16):
                rows[2][i, pl.ds(j * 16, 16)] = z
        for k in range(ZR // CH):
            pltpu.sync_copy(rows[2], acc_sh.at[pl.ds(s_ax * ZR + k * CH, CH)])
        plsc.subcore_barrier()

        @pl.loop(0, NITER)
        def main_loop(ci):
            for b in range(NMB):
                ch = ci * NMB + b
                rb = b % NRB
                wait_gather(b, rb)
                scale(b, rb)
                start_scatter(b, rb)
                rn = (rb + 2) % NRB
                mp = (b + 5) % NMB
                if b == 0:
                    @pl.when(ci >= 1)
                    def _():
                        wait_scatter(mp, rn)
                else:
                    wait_scatter(mp, rn)
                mf = (b + 4) % NMB
                if b <= 1:
                    start_meta(ch + 4, mf)
                else:
                    @pl.when(ci <= NITER - 2)
                    def _():
                        start_meta(ch + 4, mf)
                mg = (b + 2) % NMB
                if b <= 3:
                    wait_meta(ch + 2, mg)
                    start_gather(mg, rn)
                else:
                    @pl.when(ci <= NITER - 2)
                    def _():
                        wait_meta(ch + 2, mg)
                        start_gather(mg, rn)

        wait_scatter((NCHUNK - 1) % NMB, (NCHUNK - 1) % NRB)

        plsc.subcore_barrier()
        pltpu.sync_copy(acc_sh.at[pl.ds(s_ax * ZR, ZR)],
                        out_hbm.at[pl.ds(c_ax * N_PAD + s_ax * ZR, ZR)])

    return sc_kernel(x, src, dst, vals)


def _tc_epilogue(acc0, acc1, W, b2d):
    BLK = 512

    def body(a0_ref, a1_ref, w_ref, b_ref, o_ref):
        a = a0_ref[...] + a1_ref[...]
        y = jnp.dot(a, w_ref[...], preferred_element_type=jnp.float32)
        o_ref[...] = jnp.maximum(y + b_ref[...], 0.0)

    return pl.pallas_call(
        body,
        grid=(N_PAD // BLK,),
        in_specs=[
            pl.BlockSpec((BLK, D), lambda i: (i, 0)),
            pl.BlockSpec((BLK, D), lambda i: (i, 0)),
            pl.BlockSpec((D, D), lambda i: (0, 0)),
            pl.BlockSpec((1, D), lambda i: (0, 0)),
        ],
        out_specs=pl.BlockSpec((BLK, D), lambda i: (i, 0)),
        out_shape=jax.ShapeDtypeStruct((N_PAD, D), jnp.float32),
    )(acc0, acc1, W, b2d)


def kernel(x, adj_indices, adj_values, W, b):
    dst = adj_indices[0]
    src = adj_indices[1]
    pad = E_PAD - E
    src_p = jnp.pad(src, (0, pad))
    dst_p = jnp.pad(dst, (0, pad))
    val_p = jnp.pad(adj_values, (0, pad))
    acc = _sc_sparse_accumulate(x, src_p, dst_p, val_p)
    out = _tc_epilogue(acc[:N_PAD], acc[N_PAD:], W, b.reshape(1, D))
    return out[:N]


# speedup vs baseline: 6.5858x; 1.0560x over previous
import functools

import jax
import jax.numpy as jnp
from jax import lax
from jax.experimental import pallas as pl
from jax.experimental.pallas import tpu as pltpu
from jax.experimental.pallas import tpu_sc as plsc

N = 10000
E = 320000
D = 128
N_PAD = 10240

NC = 2
NS = 16
NW = NC * NS
CH = 112
NRB = 3
NMB = 6
NCHUNK = 90
EP = NCHUNK * CH
E_PAD = NW * EP
ZR = N_PAD // NS
NITER = NCHUNK // NMB


def _sc_sparse_accumulate(x, src, dst, vals):
    mesh = plsc.VectorSubcoreMesh(core_axis_name="c", subcore_axis_name="s")

    scratch = (
        [pltpu.VMEM((CH,), jnp.int32) for _ in range(NMB)]
        + [pltpu.VMEM((CH,), jnp.int32) for _ in range(NMB)]
        + [pltpu.VMEM((CH,), jnp.float32) for _ in range(NMB)]
        + [pltpu.VMEM((CH, D), jnp.float32) for _ in range(NRB)]
        + [pltpu.VMEM_SHARED((N_PAD, D), jnp.float32)]
        + [pltpu.SemaphoreType.DMA for _ in range(NMB)]
        + [pltpu.SemaphoreType.DMA for _ in range(NRB)]
        + [pltpu.SemaphoreType.DMA for _ in range(NRB)]
    )

    @functools.partial(
        pl.kernel,
        out_type=jax.ShapeDtypeStruct((NC * N_PAD, D), jnp.float32),
        mesh=mesh,
        scratch_types=scratch,
    )
    def sc_kernel(x_hbm, src_hbm, dst_hbm, val_hbm, out_hbm, *refs):
        msrc = refs[0:NMB]
        mdst = refs[NMB:2 * NMB]
        mval = refs[2 * NMB:3 * NMB]
        rows = refs[3 * NMB:3 * NMB + NRB]
        acc_sh = refs[3 * NMB + NRB]
        msem = refs[3 * NMB + NRB + 1:3 * NMB + NRB + 1 + NMB]
        gsem = refs[3 * NMB + NRB + 1 + NMB:3 * NMB + NRB + 1 + NMB + NRB]
        ssem = refs[3 * NMB + NRB + 1 + NMB + NRB:]

        c_ax = lax.axis_index("c")
        s_ax = lax.axis_index("s")
        wid = s_ax * NC + c_ax
        base = wid * EP

        def start_meta(ch, m):
            off = base + ch * CH
            pltpu.async_copy(src_hbm.at[pl.ds(off, CH)], msrc[m], msem[m])
            pltpu.async_copy(dst_hbm.at[pl.ds(off, CH)], mdst[m], msem[m])
            pltpu.async_copy(val_hbm.at[pl.ds(off, CH)], mval[m], msem[m])

        def wait_meta(ch, m):
            off = base + ch * CH
            pltpu.make_async_copy(src_hbm.at[pl.ds(off, CH)], msrc[m],
                                  msem[m]).wait()
            pltpu.make_async_copy(dst_hbm.at[pl.ds(off, CH)], mdst[m],
                                  msem[m]).wait()
            pltpu.make_async_copy(val_hbm.at[pl.ds(off, CH)], mval[m],
                                  msem[m]).wait()

        def start_gather(m, r):
            pltpu.async_copy(x_hbm.at[msrc[m]], rows[r], gsem[r])

        def wait_gather(m, r):
            pltpu.make_async_copy(x_hbm.at[msrc[m]], rows[r], gsem[r]).wait()

        def start_scatter(m, r):
            pltpu.async_copy(rows[r], acc_sh.at[mdst[m]], ssem[r], add=True)

        def wait_scatter(m, r):
            pltpu.make_async_copy(rows[r], acc_sh.at[mdst[m]], ssem[r]).wait()

        def scale(m, r):
            @pl.loop(0, CH // 16)
            def scale_group(g):
                vvec = mval[m][pl.ds(g * 16, 16)]
                for l in range(16):
                    lane = jnp.full((16,), l, jnp.int32)
                    vv = vvec.at[lane].get(mode="promise_in_bounds")
                    e = g * 16 + l
                    for j in range(D // 16):
                        sl = pl.ds(j * 16, 16)
                        rows[r][e, sl] = rows[r][e, sl] * vv

        for ch in range(4):
            start_meta(ch, ch)
        for ch in range(2):
            wait_meta(ch, ch)
            start_gather(ch, ch)

        @pl.loop(0, CH)
        def zero_body(i):
            z = jnp.zeros((16,), jnp.float32)
            for j in range(D // 16):
                rows[2][i, pl.ds(j * 16, 16)] = z
        for k in range(ZR // CH):
            pltpu.sync_copy(rows[2], acc_sh.at[pl.ds(s_ax * ZR + k * CH, CH)])
        zrem = ZR % CH
        if zrem:
            pltpu.sync_copy(
                rows[2].at[pl.ds(0, zrem)],
                acc_sh.at[pl.ds(s_ax * ZR + (ZR // CH) * CH, zrem)])
        plsc.subcore_barrier()

        @pl.loop(0, NITER)
        def main_loop(ci):
            for b in range(NMB):
                ch = ci * NMB + b
                rb = b % NRB
                wait_gather(b, rb)
                scale(b, rb)
                start_scatter(b, rb)
                rn = (rb + 2) % NRB
                mp = (b + 5) % NMB
                if b == 0:
                    @pl.when(ci >= 1)
                    def _():
                        wait_scatter(mp, rn)
                else:
                    wait_scatter(mp, rn)
                mf = (b + 4) % NMB
                if b <= 1:
                    start_meta(ch + 4, mf)
                else:
                    @pl.when(ci <= NITER - 2)
                    def _():
                        start_meta(ch + 4, mf)
                mg = (b + 2) % NMB
                if b <= 3:
                    wait_meta(ch + 2, mg)
                    start_gather(mg, rn)
                else:
                    @pl.when(ci <= NITER - 2)
                    def _():
                        wait_meta(ch + 2, mg)
                        start_gather(mg, rn)

        wait_scatter((NCHUNK - 1) % NMB, (NCHUNK - 1) % NRB)

        plsc.subcore_barrier()
        pltpu.sync_copy(acc_sh.at[pl.ds(s_ax * ZR, ZR)],
                        out_hbm.at[pl.ds(c_ax * N_PAD + s_ax * ZR, ZR)])

    return sc_kernel(x, src, dst, vals)


def _tc_epilogue(acc0, acc1, W, b2d):
    BLK = 512

    def body(a0_ref, a1_ref, w_ref, b_ref, o_ref):
        a = a0_ref[...] + a1_ref[...]
        y = jnp.dot(a, w_ref[...], preferred_element_type=jnp.float32)
        o_ref[...] = jnp.maximum(y + b_ref[...], 0.0)

    return pl.pallas_call(
        body,
        grid=(N_PAD // BLK,),
        in_specs=[
            pl.BlockSpec((BLK, D), lambda i: (i, 0)),
            pl.BlockSpec((BLK, D), lambda i: (i, 0)),
            pl.BlockSpec((D, D), lambda i: (0, 0)),
            pl.BlockSpec((1, D), lambda i: (0, 0)),
        ],
        out_specs=pl.BlockSpec((BLK, D), lambda i: (i, 0)),
        out_shape=jax.ShapeDtypeStruct((N_PAD, D), jnp.float32),
    )(acc0, acc1, W, b2d)


def kernel(x, adj_indices, adj_values, W, b):
    dst = adj_indices[0]
    src = adj_indices[1]
    pad = E_PAD - E
    src_p = jnp.pad(src, (0, pad))
    dst_p = jnp.pad(dst, (0, pad))
    val_p = jnp.pad(adj_values, (0, pad))
    acc = _sc_sparse_accumulate(x, src_p, dst_p, val_p)
    out = _tc_epilogue(acc[:N_PAD], acc[N_PAD:], W, b.reshape(1, D))
    return out[:N]
